# Initial kernel scaffold; baseline (speedup 1.0000x reference)
import functools

import jax
import jax.numpy as jnp
from jax import lax
from jax.experimental import pallas as pl
from jax.experimental.pallas import tpu as pltpu
from jax.experimental.pallas import tpu_sc as plsc

N = 10000
E = 320000
D = 128

NC = 2
NS = 16
NW = NC * NS

NP = 10240
ROWS_W = NP // NW
EPW = E // NW
K = 80
NCHUNK = EPW // K
NROWS_T = N // NS

_mesh = plsc.VectorSubcoreMesh(core_axis_name="c", subcore_axis_name="s",
                               num_cores=NC, num_subcores=NS)


def _wid():
    return lax.axis_index("s") * NC + lax.axis_index("c")


@functools.partial(
    pl.kernel,
    out_type=jax.ShapeDtypeStruct((NP, D), jnp.float32),
    mesh=_mesh,
    scratch_types=[
        pltpu.VMEM((K,), jnp.int32),
        pltpu.VMEM((K, D), jnp.float32),
        pltpu.SemaphoreType.DMA,
    ],
)
def _emb_gather(xp_hbm, emb_hbm, h_out, idx_v, rows_v, sem):
    base = _wid() * ROWS_W

    def body(i, _):
        off = base + i * K
        pltpu.sync_copy(xp_hbm.at[pl.ds(off, K)], idx_v)
        pltpu.async_copy(emb_hbm.at[idx_v], rows_v, sem).wait()
        pltpu.sync_copy(rows_v, h_out.at[pl.ds(off, K)])
        return 0

    lax.fori_loop(0, ROWS_W // K, body, 0)


def _make_segsum(with_counts):
    out_type = [jax.ShapeDtypeStruct((NC, N, D), jnp.float32)]
    scratch = [
        pltpu.VMEM((K,), jnp.int32),
        pltpu.VMEM((K,), jnp.int32),
        pltpu.VMEM((K, D), jnp.float32),
        pltpu.VMEM_SHARED((N, D), jnp.float32),
        pltpu.SemaphoreType.DMA,
    ]
    if with_counts:
        out_type.append(jax.ShapeDtypeStruct((NC, N, 16), jnp.float32))
        scratch += [
            pltpu.VMEM((K, 16), jnp.float32),
            pltpu.VMEM_SHARED((N, 16), jnp.float32),
        ]

    def body(h_hbm, src_hbm, dst_hbm, z128_hbm, *rest):
        if with_counts:
            (z16_hbm, s_out, c_out, sidx_v, didx_v, rows_v, acc_sh, sem,
             ones_v, cnt_sh) = rest
        else:
            s_out, sidx_v, didx_v, rows_v, acc_sh, sem = rest
        core = lax.axis_index("c")
        sub = lax.axis_index("s")
        rbase = sub * NROWS_T
        pltpu.sync_copy(z128_hbm.at[pl.ds(rbase, NROWS_T)],
                        acc_sh.at[pl.ds(rbase, NROWS_T)])
        if with_counts:
            pltpu.sync_copy(z16_hbm.at[pl.ds(rbase, NROWS_T)],
                            cnt_sh.at[pl.ds(rbase, NROWS_T)])
            for r in range(K):
                ones_v[r, :] = jnp.full((16,), 1.0, jnp.float32)
        plsc.subcore_barrier()

        ebase = _wid() * EPW

        def loop(i, _):
            off = ebase + i * K
            pltpu.sync_copy(src_hbm.at[pl.ds(off, K)], sidx_v)
            pltpu.sync_copy(dst_hbm.at[pl.ds(off, K)], didx_v)
            pltpu.async_copy(h_hbm.at[sidx_v], rows_v, sem).wait()
            pltpu.sync_copy(rows_v, acc_sh.at[didx_v], add=True)
            if with_counts:
                pltpu.sync_copy(ones_v, cnt_sh.at[didx_v], add=True)
            return 0

        lax.fori_loop(0, NCHUNK, loop, 0)
        plsc.subcore_barrier()
        pltpu.sync_copy(acc_sh.at[pl.ds(rbase, NROWS_T)],
                        s_out.at[core, pl.ds(rbase, NROWS_T)])
        if with_counts:
            pltpu.sync_copy(cnt_sh.at[pl.ds(rbase, NROWS_T)],
                            c_out.at[core, pl.ds(rbase, NROWS_T)])

    return pl.kernel(body, out_type=out_type, mesh=_mesh,
                     scratch_types=scratch)


_segsum_counts = _make_segsum(True)
_segsum = _make_segsum(False)


def _make_combine(relu):
    R = 1000
    grid = N // R

    def body(s_ref, c_ref, h_ref, wl_ref, b_ref, wr_ref, o_ref):
        cnt = (c_ref[0] + c_ref[1])[:, 0:1]
        inv = 1.0 / jnp.maximum(cnt, 1.0)
        agg = (s_ref[0] + s_ref[1]) * inv
        dn = (((1,), (1,)), ((), ()))
        y = (lax.dot_general(agg, wl_ref[...], dn,
                             preferred_element_type=jnp.float32)
             + b_ref[...]
             + lax.dot_general(h_ref[...], wr_ref[...], dn,
                               preferred_element_type=jnp.float32))
        o_ref[...] = jnp.maximum(y, 0.0) if relu else y

    return pl.pallas_call(
        body,
        grid=(grid,),
        in_specs=[
            pl.BlockSpec((NC, R, D), lambda i: (0, i, 0)),
            pl.BlockSpec((NC, R, 16), lambda i: (0, i, 0)),
            pl.BlockSpec((R, D), lambda i: (i, 0)),
            pl.BlockSpec((D, D), lambda i: (0, 0)),
            pl.BlockSpec((1, D), lambda i: (0, 0)),
            pl.BlockSpec((D, D), lambda i: (0, 0)),
        ],
        out_specs=pl.BlockSpec((R, D), lambda i: (i, 0)),
        out_shape=jax.ShapeDtypeStruct((N, D), jnp.float32),
    )


_combine_relu = _make_combine(True)
_combine_lin = _make_combine(False)


@functools.partial(
    pl.kernel,
    out_type=jax.ShapeDtypeStruct((E,), jnp.float32),
    mesh=_mesh,
    scratch_types=[
        pltpu.VMEM((K,), jnp.int32),
        pltpu.VMEM((K,), jnp.int32),
        pltpu.VMEM((K, D), jnp.float32),
        pltpu.VMEM((K, D), jnp.float32),
        pltpu.VMEM((K,), jnp.float32),
        pltpu.SemaphoreType.DMA,
        pltpu.SemaphoreType.DMA,
    ],
)
def _decode(z_hbm, e0_hbm, e1_hbm, out_hbm, i0_v, i1_v, a_v, b_v, o_v,
            sem0, sem1):
    ebase = _wid() * EPW
    lane = lax.iota(jnp.int32, 16)

    def loop(i, _):
        off = ebase + i * K
        pltpu.sync_copy(e0_hbm.at[pl.ds(off, K)], i0_v)
        pltpu.sync_copy(e1_hbm.at[pl.ds(off, K)], i1_v)
        cp0 = pltpu.async_copy(z_hbm.at[i0_v], a_v, sem0)
        cp1 = pltpu.async_copy(z_hbm.at[i1_v], b_v, sem1)
        cp0.wait()
        cp1.wait()

        def group(g, _):
            res = jnp.zeros((16,), jnp.float32)
            for l in range(16):
                e = g * 16 + l
                acc = a_v[e, pl.ds(0, 16)] * b_v[e, pl.ds(0, 16)]
                for c in range(1, 8):
                    acc += a_v[e, pl.ds(c * 16, 16)] * b_v[e, pl.ds(c * 16, 16)]
                res = jnp.where(lane == l, jnp.sum(acc), res)
            o_v[pl.ds(g * 16, 16)] = res
            return 0

        lax.fori_loop(0, K // 16, group, 0)
        pltpu.sync_copy(o_v, out_hbm.at[pl.ds(off, K)])
        return 0

    lax.fori_loop(0, NCHUNK, loop, 0)


def kernel(x, edge_index, edge_position, emb_table, W1l, b1, W1r, W2l, b2,
           W2r):
    src = edge_index[0]
    dst = edge_index[1]
    ep0 = edge_position[0]
    ep1 = edge_position[1]
    xp = jnp.concatenate([x[:, 0], jnp.zeros((NP - N,), jnp.int32)])
    z128 = jnp.zeros((N, D), jnp.float32)
    z16 = jnp.zeros((N, 16), jnp.float32)
    b1r = b1.reshape(1, D)
    b2r = b2.reshape(1, D)

    h = _emb_gather(xp, emb_table)
    s1, c1 = _segsum_counts(h, src, dst, z128, z16)
    h1 = _combine_relu(s1, c1, h[:N], W1l, b1r, W1r)
    (s2,) = _segsum(h1, src, dst, z128)
    z = _combine_lin(s2, c1, h1, W2l, b2r, W2r)
    return _decode(z, ep0, ep1)


# speedup vs baseline: 4.3765x; 4.3765x over previous
import functools

import jax
import jax.numpy as jnp
from jax import lax
from jax.experimental import pallas as pl
from jax.experimental.pallas import tpu as pltpu
from jax.experimental.pallas import tpu_sc as plsc

N = 10000
E = 320000
D = 128

NC = 2
NS = 16
NW = NC * NS

NP = 10240
ROWS_W = NP // NW
EPW = E // NW
K = 80
NCHUNK = EPW // K
NPA = 10240
NROWS_T = NPA // NS

_mesh = plsc.VectorSubcoreMesh(core_axis_name="c", subcore_axis_name="s",
                               num_cores=NC, num_subcores=NS)
_cp = pltpu.CompilerParams(needs_layout_passes=False)


def _wid():
    return lax.axis_index("s") * NC + lax.axis_index("c")


@functools.partial(
    pl.kernel,
    out_type=[jax.ShapeDtypeStruct((NP, D), jnp.float32),
              jax.ShapeDtypeStruct((NW, NPA), jnp.float32)],
    mesh=_mesh,
    compiler_params=_cp,
    scratch_types=[
        pltpu.VMEM((K,), jnp.int32),
        pltpu.VMEM((K, D), jnp.float32),
        pltpu.VMEM((K,), jnp.int32),
        pltpu.VMEM((NPA,), jnp.float32),
        pltpu.SemaphoreType.DMA,
    ],
)
def _emb_gather(xp_hbm, emb_hbm, dst_hbm, zn_hbm, h_out, c_out,
                idx_v, rows_v, didx_v, cnt_v, sem):
    w = _wid()
    base = w * ROWS_W

    def body(i, _):
        off = base + i * K
        pltpu.sync_copy(xp_hbm.at[pl.ds(off, K)], idx_v)
        pltpu.async_copy(emb_hbm.at[idx_v], rows_v, sem).wait()
        pltpu.sync_copy(rows_v, h_out.at[pl.ds(off, K)])
        return 0

    lax.fori_loop(0, ROWS_W // K, body, 0)

    pltpu.sync_copy(zn_hbm, cnt_v)
    ones = jnp.full((16,), 1.0, jnp.float32)
    ebase = w * EPW

    def cbody(i, _):
        off = ebase + i * K
        pltpu.sync_copy(dst_hbm.at[pl.ds(off, K)], didx_v)
        for j in range(K // 16):
            plsc.addupdate_scatter(cnt_v, [didx_v[pl.ds(j * 16, 16)]], ones)
        return 0

    lax.fori_loop(0, NCHUNK, cbody, 0)
    pltpu.sync_copy(cnt_v, c_out.at[w])


@functools.partial(
    pl.kernel,
    out_type=jax.ShapeDtypeStruct((NC, NPA, D), jnp.float32),
    mesh=_mesh,
    compiler_params=_cp,
    scratch_types=[
        pltpu.VMEM((K,), jnp.int32),
        pltpu.VMEM((K,), jnp.int32),
        pltpu.VMEM((K, D), jnp.float32),
        pltpu.VMEM_SHARED((NPA, D), jnp.float32),
        pltpu.SemaphoreType.DMA,
    ],
)
def _segsum(h_hbm, src_hbm, dst_hbm, z128_hbm, s_out,
            sidx_v, didx_v, rows_v, acc_sh, sem):
    core = lax.axis_index("c")
    sub = lax.axis_index("s")
    rbase = sub * NROWS_T
    pltpu.sync_copy(z128_hbm.at[pl.ds(rbase, NROWS_T)],
                    acc_sh.at[pl.ds(rbase, NROWS_T)])
    plsc.subcore_barrier()

    ebase = _wid() * EPW

    def loop(i, _):
        off = ebase + i * K
        pltpu.sync_copy(src_hbm.at[pl.ds(off, K)], sidx_v)
        pltpu.sync_copy(dst_hbm.at[pl.ds(off, K)], didx_v)
        pltpu.async_copy(h_hbm.at[sidx_v], rows_v, sem).wait()
        pltpu.sync_copy(rows_v, acc_sh.at[didx_v], add=True)
        return 0

    lax.fori_loop(0, NCHUNK, loop, 0)
    plsc.subcore_barrier()
    pltpu.sync_copy(acc_sh.at[pl.ds(rbase, NROWS_T)],
                    s_out.at[core, pl.ds(rbase, NROWS_T)])


def _make_combine(relu):
    R = 1280
    grid = NPA // R

    def body(s_ref, c_ref, h_ref, wl_ref, b_ref, wr_ref, o_ref):
        cnt = jnp.sum(c_ref[...], axis=0)[:, None]
        inv = 1.0 / jnp.maximum(cnt, 1.0)
        agg = (s_ref[0] + s_ref[1]) * inv
        dn = (((1,), (1,)), ((), ()))
        y = (lax.dot_general(agg, wl_ref[...], dn,
                             preferred_element_type=jnp.float32)
             + b_ref[...]
             + lax.dot_general(h_ref[...], wr_ref[...], dn,
                               preferred_element_type=jnp.float32))
        o_ref[...] = jnp.maximum(y, 0.0) if relu else y

    return pl.pallas_call(
        body,
        grid=(grid,),
        in_specs=[
            pl.BlockSpec((NC, R, D), lambda i: (0, i, 0)),
            pl.BlockSpec((NW, R), lambda i: (0, i)),
            pl.BlockSpec((R, D), lambda i: (i, 0)),
            pl.BlockSpec((D, D), lambda i: (0, 0)),
            pl.BlockSpec((1, D), lambda i: (0, 0)),
            pl.BlockSpec((D, D), lambda i: (0, 0)),
        ],
        out_specs=pl.BlockSpec((R, D), lambda i: (i, 0)),
        out_shape=jax.ShapeDtypeStruct((NPA, D), jnp.float32),
    )


_combine_relu = _make_combine(True)
_combine_lin = _make_combine(False)


@functools.partial(
    pl.kernel,
    out_type=jax.ShapeDtypeStruct((E,), jnp.float32),
    mesh=_mesh,
    compiler_params=_cp,
    scratch_types=[
        pltpu.VMEM((K,), jnp.int32),
        pltpu.VMEM((K,), jnp.int32),
        pltpu.VMEM((K, D), jnp.float32),
        pltpu.VMEM((K, D), jnp.float32),
        pltpu.VMEM((K,), jnp.float32),
        pltpu.VMEM((16, 16), jnp.float32),
        pltpu.SemaphoreType.DMA,
        pltpu.SemaphoreType.DMA,
    ],
)
def _decode(z_hbm, e0_hbm, e1_hbm, out_hbm, i0_v, i1_v, a_v, b_v, o_v, t_v,
            sem0, sem1):
    ebase = _wid() * EPW
    lane = lax.iota(jnp.int32, 16)

    def loop(i, _):
        off = ebase + i * K
        pltpu.sync_copy(e0_hbm.at[pl.ds(off, K)], i0_v)
        pltpu.sync_copy(e1_hbm.at[pl.ds(off, K)], i1_v)
        cp0 = pltpu.async_copy(z_hbm.at[i0_v], a_v, sem0)
        cp1 = pltpu.async_copy(z_hbm.at[i1_v], b_v, sem1)
        cp0.wait()
        cp1.wait()

        def group(g, _):
            for l in range(16):
                e = g * 16 + l
                acc = a_v[e, pl.ds(0, 16)] * b_v[e, pl.ds(0, 16)]
                for c in range(1, 8):
                    acc += a_v[e, pl.ds(c * 16, 16)] * b_v[e, pl.ds(c * 16, 16)]
                plsc.store_scatter(t_v, [lane, jnp.full((16,), l, jnp.int32)],
                                   acc)
            res = t_v[0, :]
            for r in range(1, 16):
                res = res + t_v[r, :]
            o_v[pl.ds(g * 16, 16)] = res
            return 0

        lax.fori_loop(0, K // 16, group, 0)
        pltpu.sync_copy(o_v, out_hbm.at[pl.ds(off, K)])
        return 0

    lax.fori_loop(0, NCHUNK, loop, 0)


def kernel(x, edge_index, edge_position, emb_table, W1l, b1, W1r, W2l, b2,
           W2r):
    src = edge_index[0]
    dst = edge_index[1]
    ep0 = edge_position[0]
    ep1 = edge_position[1]
    xp = jnp.concatenate([x[:, 0], jnp.zeros((NP - N,), jnp.int32)])
    z128 = jnp.zeros((NPA, D), jnp.float32)
    zn = jnp.zeros((NPA,), jnp.float32)
    b1r = b1.reshape(1, D)
    b2r = b2.reshape(1, D)

    h, c32 = _emb_gather(xp, emb_table, dst, zn)
    s1 = _segsum(h, src, dst, z128)
    h1 = _combine_relu(s1, c32, h, W1l, b1r, W1r)
    s2 = _segsum(h1, src, dst, z128)
    z = _combine_lin(s2, c32, h1, W2l, b2r, W2r)
    return _decode(z, ep0, ep1)


# speedup vs baseline: 8.7281x; 1.9943x over previous
import functools

import jax
import jax.numpy as jnp
from jax import lax
from jax.experimental import pallas as pl
from jax.experimental.pallas import tpu as pltpu
from jax.experimental.pallas import tpu_sc as plsc

N = 10000
E = 320000
D = 128

NC = 2
NS = 16
NW = NC * NS

NP = 10240
ROWS_W = NP // NW
EPW = E // NW
K = 80
NCHUNK = EPW // K
NPA = 10240
NROWS_T = NPA // NS

_mesh = plsc.VectorSubcoreMesh(core_axis_name="c", subcore_axis_name="s",
                               num_cores=NC, num_subcores=NS)
_cp = pltpu.CompilerParams(needs_layout_passes=False)


def _wid():
    return lax.axis_index("s") * NC + lax.axis_index("c")


@functools.partial(
    pl.kernel,
    out_type=[jax.ShapeDtypeStruct((NP, D), jnp.float32),
              jax.ShapeDtypeStruct((NW, NPA), jnp.float32)],
    mesh=_mesh,
    compiler_params=_cp,
    scratch_types=[
        pltpu.VMEM((K,), jnp.int32),
        pltpu.VMEM((K, D), jnp.float32),
        pltpu.VMEM((NCHUNK, K), jnp.int32),
        pltpu.VMEM((NPA,), jnp.float32),
        pltpu.SemaphoreType.DMA,
    ],
)
def _emb_gather(xp_hbm, emb_hbm, dst3_hbm, zn_hbm, h_out, c_out,
                idx_v, rows_v, didx_all, cnt_v, sem):
    w = _wid()
    base = w * ROWS_W

    def body(i, _):
        off = base + i * K
        pltpu.sync_copy(xp_hbm.at[pl.ds(off, K)], idx_v)
        pltpu.async_copy(emb_hbm.at[idx_v], rows_v, sem).wait()
        pltpu.sync_copy(rows_v, h_out.at[pl.ds(off, K)])
        return 0

    lax.fori_loop(0, ROWS_W // K, body, 0)

    pltpu.sync_copy(zn_hbm, cnt_v)
    pltpu.sync_copy(dst3_hbm.at[w], didx_all)
    ones = jnp.full((16,), 1.0, jnp.float32)

    def cbody(i, _):
        for j in range(K // 16):
            plsc.addupdate_scatter(cnt_v, [didx_all[i, pl.ds(j * 16, 16)]],
                                   ones)
        return 0

    lax.fori_loop(0, NCHUNK, cbody, 0)
    pltpu.sync_copy(cnt_v, c_out.at[w])


@functools.partial(
    pl.kernel,
    out_type=jax.ShapeDtypeStruct((NC, NPA, D), jnp.float32),
    mesh=_mesh,
    compiler_params=_cp,
    scratch_types=[
        pltpu.VMEM((K,), jnp.int32),
        pltpu.VMEM((K,), jnp.int32),
        pltpu.VMEM((NCHUNK, K), jnp.int32),
        pltpu.VMEM((K, D), jnp.float32),
        pltpu.VMEM((K, D), jnp.float32),
        pltpu.VMEM_SHARED((NPA, D), jnp.float32),
        pltpu.SemaphoreType.DMA,
        pltpu.SemaphoreType.DMA,
        pltpu.SemaphoreType.DMA,
        pltpu.SemaphoreType.DMA,
        pltpu.SemaphoreType.DMA,
        pltpu.SemaphoreType.DMA,
    ],
)
def _segsum(h_hbm, src3_hbm, dst3_hbm, z128_hbm, s_out,
            sidx0, sidx1, didx_all, rows0, rows1, acc_sh,
            gsem0, gsem1, ssem0, ssem1, isem0, isem1):
    core = lax.axis_index("c")
    sub = lax.axis_index("s")
    rbase = sub * NROWS_T
    rows = (rows0, rows1)
    sidx = (sidx0, sidx1)
    gsem = (gsem0, gsem1)
    ssem = (ssem0, ssem1)
    isem = (isem0, isem1)
    pltpu.sync_copy(z128_hbm.at[pl.ds(rbase, NROWS_T)],
                    acc_sh.at[pl.ds(rbase, NROWS_T)])
    w = _wid()
    pltpu.sync_copy(dst3_hbm.at[w], didx_all)
    plsc.subcore_barrier()

    def i_start(ci, b):
        pltpu.async_copy(src3_hbm.at[w].at[ci], sidx[b], isem[b])

    def i_wait(ci, b):
        pltpu.make_async_copy(src3_hbm.at[w].at[ci], sidx[b], isem[b]).wait()

    def g_start(ci, b):
        pltpu.async_copy(h_hbm.at[sidx[b]], rows[b], gsem[b])

    def g_wait(ci, b):
        pltpu.make_async_copy(h_hbm.at[sidx[b]], rows[b], gsem[b]).wait()

    def s_start(ci, b):
        pltpu.async_copy(rows[b], acc_sh.at[didx_all.at[ci]], ssem[b],
                         add=True)

    def s_wait(ci, b):
        pltpu.make_async_copy(rows[b], acc_sh.at[didx_all.at[ci]],
                              ssem[b]).wait()

    i_start(0, 0)
    i_wait(0, 0)
    g_start(0, 0)
    i_start(1, 1)

    def pair(p, _):
        for b in range(2):
            ci = 2 * p + b
            g_wait(ci, b)

            @pl.when(ci >= 1)
            def _():
                s_wait(ci - 1, 1 - b)

            i_wait(ci + 1, 1 - b)
            g_start(ci + 1, 1 - b)
            s_start(ci, b)

            @pl.when(ci + 2 < NCHUNK)
            def _():
                i_start(ci + 2, b)
        return 0

    lax.fori_loop(0, (NCHUNK - 1) // 2, pair, 0)
    last = NCHUNK - 1
    g_wait(last, 0)
    s_wait(last - 1, 1)
    s_start(last, 0)
    s_wait(last, 0)

    plsc.subcore_barrier()
    pltpu.sync_copy(acc_sh.at[pl.ds(rbase, NROWS_T)],
                    s_out.at[core, pl.ds(rbase, NROWS_T)])


def _make_combine(relu):
    R = 1280
    grid = NPA // R

    def body(s_ref, c_ref, h_ref, wl_ref, b_ref, wr_ref, o_ref):
        cnt = jnp.sum(c_ref[...], axis=0)[:, None]
        inv = 1.0 / jnp.maximum(cnt, 1.0)
        agg = (s_ref[0] + s_ref[1]) * inv
        dn = (((1,), (1,)), ((), ()))
        y = (lax.dot_general(agg, wl_ref[...], dn,
                             preferred_element_type=jnp.float32)
             + b_ref[...]
             + lax.dot_general(h_ref[...], wr_ref[...], dn,
                               preferred_element_type=jnp.float32))
        o_ref[...] = jnp.maximum(y, 0.0) if relu else y

    return pl.pallas_call(
        body,
        grid=(grid,),
        in_specs=[
            pl.BlockSpec((NC, R, D), lambda i: (0, i, 0)),
            pl.BlockSpec((NW, R), lambda i: (0, i)),
            pl.BlockSpec((R, D), lambda i: (i, 0)),
            pl.BlockSpec((D, D), lambda i: (0, 0)),
            pl.BlockSpec((1, D), lambda i: (0, 0)),
            pl.BlockSpec((D, D), lambda i: (0, 0)),
        ],
        out_specs=pl.BlockSpec((R, D), lambda i: (i, 0)),
        out_shape=jax.ShapeDtypeStruct((NPA, D), jnp.float32),
    )


_combine_relu = _make_combine(True)
_combine_lin = _make_combine(False)


@functools.partial(
    pl.kernel,
    out_type=jax.ShapeDtypeStruct((E,), jnp.float32),
    mesh=_mesh,
    compiler_params=_cp,
    scratch_types=[
        pltpu.VMEM((NCHUNK, K), jnp.int32),
        pltpu.VMEM((NCHUNK, K), jnp.int32),
        pltpu.VMEM((K, D), jnp.float32),
        pltpu.VMEM((K, D), jnp.float32),
        pltpu.VMEM((K, D), jnp.float32),
        pltpu.VMEM((K, D), jnp.float32),
        pltpu.VMEM((K,), jnp.float32),
        pltpu.VMEM((K,), jnp.float32),
        pltpu.VMEM((16, 16), jnp.float32),
        pltpu.SemaphoreType.DMA,
        pltpu.SemaphoreType.DMA,
        pltpu.SemaphoreType.DMA,
        pltpu.SemaphoreType.DMA,
        pltpu.SemaphoreType.DMA,
        pltpu.SemaphoreType.DMA,
    ],
)
def _decode(z_hbm, e03_hbm, e13_hbm, out_hbm,
            i0_all, i1_all, a0_v, a1_v, b0_v, b1_v, o0_v, o1_v, t_v,
            ga0, ga1, gb0, gb1, ws0, ws1):
    w = _wid()
    ebase = w * EPW
    lane = lax.iota(jnp.int32, 16)
    av = (a0_v, a1_v)
    bv = (b0_v, b1_v)
    ov = (o0_v, o1_v)
    gas = (ga0, ga1)
    gbs = (gb0, gb1)
    wss = (ws0, ws1)

    pltpu.sync_copy(e03_hbm.at[w], i0_all)
    pltpu.sync_copy(e13_hbm.at[w], i1_all)

    def g_start(ci, b):
        pltpu.async_copy(z_hbm.at[i0_all.at[ci]], av[b], gas[b])
        pltpu.async_copy(z_hbm.at[i1_all.at[ci]], bv[b], gbs[b])

    def g_wait(ci, b):
        pltpu.make_async_copy(z_hbm.at[i0_all.at[ci]], av[b], gas[b]).wait()
        pltpu.make_async_copy(z_hbm.at[i1_all.at[ci]], bv[b], gbs[b]).wait()

    def w_start(ci, b):
        off = ebase + ci * K
        pltpu.async_copy(ov[b], out_hbm.at[pl.ds(off, K)], wss[b])

    def w_wait(ci, b):
        off = ebase + ci * K
        pltpu.make_async_copy(ov[b], out_hbm.at[pl.ds(off, K)], wss[b]).wait()

    def compute(b):
        a_v = av[b]
        b_v = bv[b]
        o_v = ov[b]

        def group(g, _):
            for l in range(16):
                e = g * 16 + l
                acc = a_v[e, pl.ds(0, 16)] * b_v[e, pl.ds(0, 16)]
                for c in range(1, 8):
                    acc += a_v[e, pl.ds(c * 16, 16)] * b_v[e, pl.ds(c * 16, 16)]
                plsc.store_scatter(t_v, [lane, jnp.full((16,), l, jnp.int32)],
                                   acc)
            res = t_v[0, :]
            for r in range(1, 16):
                res = res + t_v[r, :]
            o_v[pl.ds(g * 16, 16)] = res
            return 0

        lax.fori_loop(0, K // 16, group, 0)

    g_start(0, 0)

    def pair(p, _):
        for b in range(2):
            ci = 2 * p + b
            g_wait(ci, b)
            g_start(ci + 1, 1 - b)

            @pl.when(ci >= 2)
            def _():
                w_wait(ci - 2, b)

            compute(b)
            w_start(ci, b)
        return 0

    lax.fori_loop(0, (NCHUNK - 1) // 2, pair, 0)
    last = NCHUNK - 1
    g_wait(last, 0)
    w_wait(last - 2, 0)
    compute(0)
    w_start(last, 0)
    w_wait(last - 1, 1)
    w_wait(last, 0)


def kernel(x, edge_index, edge_position, emb_table, W1l, b1, W1r, W2l, b2,
           W2r):
    src3 = edge_index[0].reshape(NW, NCHUNK, K)
    dst3 = edge_index[1].reshape(NW, NCHUNK, K)
    e03 = edge_position[0].reshape(NW, NCHUNK, K)
    e13 = edge_position[1].reshape(NW, NCHUNK, K)
    xp = jnp.concatenate([x[:, 0], jnp.zeros((NP - N,), jnp.int32)])
    z128 = jnp.zeros((NPA, D), jnp.float32)
    zn = jnp.zeros((NPA,), jnp.float32)
    b1r = b1.reshape(1, D)
    b2r = b2.reshape(1, D)

    h, c32 = _emb_gather(xp, emb_table, dst3, zn)
    s1 = _segsum(h, src3, dst3, z128)
    h1 = _combine_relu(s1, c32, h, W1l, b1r, W1r)
    s2 = _segsum(h1, src3, dst3, z128)
    z = _combine_lin(s2, c32, h1, W2l, b2r, W2r)
    return _decode(z, e03, e13)
